# baseline (speedup 1.0000x reference)
import functools
import jax
import jax.numpy as jnp
from jax import lax
from jax.experimental import pallas as pl
from jax.experimental.pallas import tpu as pltpu
from jax.experimental.pallas import tpu_sc as plsc

B = 2
T = 2048
D = 2048
E = 8
H = 8
N = T // E
DH = D // H
SCALE = D ** (-0.5)
L = 16


def _route_scan_body(vals_hbm, order_hbm, probs_hbm, perm_hbm,
                     vals_v, order_v, probs_v, claimed_v, perm_v):
    b = lax.axis_index("c")
    sid = lax.axis_index("s")

    @pl.when(sid == 0)
    def _():
        pltpu.sync_copy(vals_hbm.at[b], vals_v)
        pltpu.sync_copy(order_hbm.at[b], order_v)
        pltpu.sync_copy(probs_hbm.at[b], probs_v)
        zeros16 = jnp.zeros((L,), jnp.int32)
        ones16 = jnp.ones((L,), jnp.int32)
        n_vec = jnp.full((L,), N, jnp.int32)

        def zbody(g, c):
            claimed_v[pl.ds(pl.multiple_of(g * L, L), L)] = zeros16
            return c

        lax.fori_loop(0, T // L, zbody, 0)

        for e in range(E):
            def body(g, cnt, e=e):
                off = pl.multiple_of(g * L, L)
                idx16 = order_v[e, pl.ds(off, L)]
                v16 = vals_v[e, pl.ds(off, L)]
                av16 = plsc.load_gather(claimed_v, [idx16])
                pos = (av16 == 0) & (v16 > 0.0)
                c = plsc.cumsum(pos.astype(jnp.int32)) + cnt
                pick = pos & (c <= n_vec)
                plsc.store_scatter(claimed_v, [idx16], ones16, mask=pick)
                plsc.store_scatter(perm_v, [c + (e * N - 1)], idx16, mask=pick)
                npos = plsc.all_reduce_population_count(pos)
                return jnp.minimum(cnt + npos, n_vec)

            cnt = lax.fori_loop(0, T // L, body, jnp.zeros((L,), jnp.int32))

            @pl.when(jnp.max(cnt) < N)
            def _(e=e, cnt=cnt):
                need = n_vec - cnt

                def zt_body(g, cz, e=e, cnt=cnt, need=need):
                    off = pl.multiple_of(g * L, L)
                    cl16 = claimed_v[pl.ds(off, L)]
                    p16 = probs_v[e, pl.ds(off, L)]
                    zt = (cl16 != 0) | (p16 == 0.0)
                    c2 = plsc.cumsum(zt.astype(jnp.int32)) + cz
                    pickz = zt & (c2 <= need)
                    tok16 = lax.iota(jnp.int32, L) + off
                    plsc.store_scatter(claimed_v, [tok16], ones16, mask=pickz)
                    plsc.store_scatter(perm_v, [c2 + cnt + (e * N - 1)], tok16,
                                       mask=pickz)
                    nz = plsc.all_reduce_population_count(zt)
                    return jnp.minimum(cz + nz, need)

                lax.fori_loop(0, T // L, zt_body, jnp.zeros((L,), jnp.int32))

        pltpu.sync_copy(perm_v, perm_hbm.at[b])


_route_scan = functools.partial(
    pl.kernel,
    out_type=jax.ShapeDtypeStruct((B, T), jnp.int32),
    mesh=plsc.VectorSubcoreMesh(core_axis_name="c", subcore_axis_name="s"),
    compiler_params=pltpu.CompilerParams(needs_layout_passes=False),
    scratch_types=[
        pltpu.VMEM((E, T), jnp.float32),
        pltpu.VMEM((E, T), jnp.int32),
        pltpu.VMEM((E, T), jnp.float32),
        pltpu.VMEM((T,), jnp.int32),
        pltpu.VMEM((T,), jnp.int32),
    ],
)(_route_scan_body)


def _greedy_route(probs):
    pt = jnp.transpose(probs, (0, 2, 1))
    iota = lax.broadcasted_iota(jnp.int32, (B, E, T), 2)
    neg_sorted, order = lax.sort((-pt, iota), dimension=2, num_keys=1,
                                 is_stable=True)
    return _route_scan(-neg_sorted, order, pt)


def _cast_body(a_ref, b_ref, c_ref, d_ref, ao_ref, bo_ref, co_ref, do_ref):
    ao_ref[...] = a_ref[...].astype(jnp.bfloat16)
    bo_ref[...] = b_ref[...].astype(jnp.bfloat16)
    co_ref[...] = c_ref[...].astype(jnp.bfloat16)
    do_ref[...] = d_ref[...].astype(jnp.bfloat16)


def _cast_weights(q_w, k_w, v_w, o_w):
    spec = pl.BlockSpec((D // 8, D), lambda i: (i, 0))
    out_sd = jax.ShapeDtypeStruct((D, D), jnp.bfloat16)
    return pl.pallas_call(
        _cast_body,
        grid=(8,),
        in_specs=[spec] * 4,
        out_specs=[spec] * 4,
        out_shape=[out_sd] * 4,
    )(q_w, k_w, v_w, o_w)


def _qkv_body(x_ref, qw_ref, kw_ref, vw_ref, qb_ref, kb_ref, vb_ref,
              lnw_ref, lnb_ref, q_ref, k_ref, v_ref):
    e = pl.program_id(1)
    xb = x_ref[0].astype(jnp.float32)
    mu = jnp.mean(xb, axis=1, keepdims=True)
    var = jnp.mean((xb - mu) ** 2, axis=1, keepdims=True)
    ln = (xb - mu) / jnp.sqrt(var + 1e-5) * lnw_ref[...] + lnb_ref[...]
    dn = (((1,), (1,)), ((), ()))
    for i in range(E):
        m = D >> i

        @pl.when(e == i)
        def _(m=m):
            ex = ln[:, :m].astype(jnp.bfloat16)
            q = jax.lax.dot_general(ex, qw_ref[:, :m], dn,
                                    preferred_element_type=jnp.float32) + qb_ref[...]
            k = jax.lax.dot_general(ex, kw_ref[:, :m], dn,
                                    preferred_element_type=jnp.float32) + kb_ref[...]
            v = jax.lax.dot_general(ex, vw_ref[:, :m], dn,
                                    preferred_element_type=jnp.float32) + vb_ref[...]
            q_ref[0] = q.astype(jnp.bfloat16)
            k_ref[0] = k.astype(jnp.bfloat16)
            v_ref[0] = v.astype(jnp.bfloat16)


def _qkv_all(xg_b, q_wb, k_wb, v_wb, q_b, k_b, v_b, ln_w, ln_b):
    xspec = pl.BlockSpec((1, N, D), lambda b, e: (b, e, 0))
    wspec = pl.BlockSpec((D, D), lambda b, e: (0, 0))
    bspec = pl.BlockSpec((D,), lambda b, e: (0,))
    ospec = pl.BlockSpec((1, N, D), lambda b, e: (b, e, 0))
    out_sd = jax.ShapeDtypeStruct((B, T, D), jnp.bfloat16)
    return pl.pallas_call(
        _qkv_body,
        grid=(B, E),
        in_specs=[xspec, wspec, wspec, wspec, bspec, bspec, bspec, bspec, bspec],
        out_specs=[ospec, ospec, ospec],
        out_shape=[out_sd, out_sd, out_sd],
    )(xg_b, q_wb, k_wb, v_wb, q_b, k_b, v_b, ln_w, ln_b)


def _attn_body(q_ref, k_ref, v_ref, o_ref):
    q = q_ref[0]
    k = k_ref[0]
    v = v_ref[0]
    s = jax.lax.dot_general(q, k, (((1,), (1,)), ((), ())),
                            preferred_element_type=jnp.float32) * SCALE
    mx = jnp.max(s, axis=1, keepdims=True)
    p = jnp.exp(s - mx)
    p = p / jnp.sum(p, axis=1, keepdims=True)
    o = jax.lax.dot_general(p.astype(jnp.bfloat16), v, (((1,), (0,)), ((), ())),
                            preferred_element_type=jnp.float32)
    o_ref[0] = o.astype(jnp.bfloat16)


def _attention(q, k, v, bq=256):
    qspec = pl.BlockSpec((1, bq, DH), lambda b, h, i: (b, i, h))
    kvspec = pl.BlockSpec((1, T, DH), lambda b, h, i: (b, 0, h))
    return pl.pallas_call(
        _attn_body,
        grid=(B, H, T // bq),
        in_specs=[qspec, kvspec, kvspec],
        out_specs=qspec,
        out_shape=jax.ShapeDtypeStruct((B, T, D), jnp.bfloat16),
    )(q, k, v)


def _oproj_body(a_ref, x_ref, ow_ref, ob_ref, o_ref):
    e = pl.program_id(1)
    ab = a_ref[0]
    xb = x_ref[0]
    dn = (((1,), (1,)), ((), ()))
    for i in range(E):
        m = D >> i

        @pl.when(e == i)
        def _(m=m):
            proj = jax.lax.dot_general(ab[:, :m], ow_ref[:m, :m], dn,
                                       preferred_element_type=jnp.float32) + ob_ref[:m]
            if m == D:
                o_ref[0] = xb + proj
            else:
                o_ref[0] = jnp.concatenate([xb[:, :m] + proj, xb[:, m:]], axis=1)


def _oproj_all(attn_g, x_g, o_wb, o_b):
    aspec = pl.BlockSpec((1, N, D), lambda b, e: (b, e, 0))
    wspec = pl.BlockSpec((D, D), lambda b, e: (0, 0))
    bspec = pl.BlockSpec((D,), lambda b, e: (0,))
    return pl.pallas_call(
        _oproj_body,
        grid=(B, E),
        in_specs=[aspec, aspec, wspec, bspec],
        out_specs=aspec,
        out_shape=jax.ShapeDtypeStruct((B, T, D), jnp.float32),
    )(attn_g, x_g, o_wb, o_b)


def kernel(x, router_prob, q_w, q_b, k_w, k_b, v_w, v_b, o_w, o_b, ln_w, ln_b):
    q_wb, k_wb, v_wb, o_wb = _cast_weights(q_w, k_w, v_w, o_w)

    perm = _greedy_route(router_prob)
    new_probs = jnp.take_along_axis(router_prob, perm[:, :, None], axis=1)
    xg = jnp.take_along_axis(x, perm[:, :, None], axis=1)

    q, k, v = _qkv_all(xg, q_wb, k_wb, v_wb, q_b, k_b, v_b, ln_w, ln_b)

    attn_out = _attention(q, k, v)

    perm2 = _greedy_route(new_probs)
    attn_g = jnp.take_along_axis(attn_out, perm2[:, :, None], axis=1)
    x_g = jnp.take_along_axis(x, perm2[:, :, None], axis=1)
    final_probs = jnp.take_along_axis(new_probs, perm2[:, :, None], axis=1)

    out = _oproj_all(attn_g, x_g, o_wb, o_b)
    return out, final_probs


# speedup vs baseline: 1.1049x; 1.1049x over previous
import functools
import jax
import jax.numpy as jnp
from jax import lax
from jax.experimental import pallas as pl
from jax.experimental.pallas import tpu as pltpu
from jax.experimental.pallas import tpu_sc as plsc

B = 2
T = 2048
D = 2048
E = 8
H = 8
N = T // E
DH = D // H
SCALE = D ** (-0.5)
L = 16


def _route_scan_body(vals_hbm, order_hbm, probs_hbm, perm_hbm,
                     vals_v, order_v, probs_v, claimed_v, perm_v):
    b = lax.axis_index("c")
    sid = lax.axis_index("s")

    @pl.when(sid == 0)
    def _():
        pltpu.sync_copy(vals_hbm.at[b], vals_v)
        pltpu.sync_copy(order_hbm.at[b], order_v)
        pltpu.sync_copy(probs_hbm.at[b], probs_v)
        zeros16 = jnp.zeros((L,), jnp.int32)
        ones16 = jnp.ones((L,), jnp.int32)
        n_vec = jnp.full((L,), N, jnp.int32)

        def zbody(g, c):
            claimed_v[pl.ds(pl.multiple_of(g * L, L), L)] = zeros16
            return c

        lax.fori_loop(0, T // L, zbody, 0)

        for e in range(E):
            def body(g, cnt, e=e):
                off = pl.multiple_of(g * L, L)
                idx16 = order_v[e, pl.ds(off, L)]
                v16 = vals_v[e, pl.ds(off, L)]
                av16 = plsc.load_gather(claimed_v, [idx16])
                pos = (av16 == 0) & (v16 < 0.0)
                c = plsc.cumsum(pos.astype(jnp.int32)) + cnt
                pick = pos & (c <= n_vec)
                plsc.store_scatter(claimed_v, [idx16], ones16, mask=pick)
                plsc.store_scatter(perm_v, [c + (e * N - 1)], idx16, mask=pick)
                npos = plsc.all_reduce_population_count(pos)
                return jnp.minimum(cnt + npos, n_vec)

            cnt = lax.fori_loop(0, T // L, body, jnp.zeros((L,), jnp.int32))

            @pl.when(jnp.max(cnt) < N)
            def _(e=e, cnt=cnt):
                need = n_vec - cnt

                def zt_body(g, cz, e=e, cnt=cnt, need=need):
                    off = pl.multiple_of(g * L, L)
                    cl16 = claimed_v[pl.ds(off, L)]
                    p16 = probs_v[e, pl.ds(off, L)]
                    zt = (cl16 != 0) | (p16 == 0.0)
                    c2 = plsc.cumsum(zt.astype(jnp.int32)) + cz
                    pickz = zt & (c2 <= need)
                    tok16 = lax.iota(jnp.int32, L) + off
                    plsc.store_scatter(claimed_v, [tok16], ones16, mask=pickz)
                    plsc.store_scatter(perm_v, [c2 + cnt + (e * N - 1)], tok16,
                                       mask=pickz)
                    nz = plsc.all_reduce_population_count(zt)
                    return jnp.minimum(cz + nz, need)

                lax.fori_loop(0, T // L, zt_body, jnp.zeros((L,), jnp.int32))

        pltpu.sync_copy(perm_v, perm_hbm.at[b])


_route_scan = functools.partial(
    pl.kernel,
    out_type=jax.ShapeDtypeStruct((B, T), jnp.int32),
    mesh=plsc.VectorSubcoreMesh(core_axis_name="c", subcore_axis_name="s"),
    compiler_params=pltpu.CompilerParams(needs_layout_passes=False),
    scratch_types=[
        pltpu.VMEM((E, T), jnp.float32),
        pltpu.VMEM((E, T), jnp.int32),
        pltpu.VMEM((E, T), jnp.float32),
        pltpu.VMEM((T,), jnp.int32),
        pltpu.VMEM((T,), jnp.int32),
    ],
)(_route_scan_body)


def _greedy_route(probs):
    npt = jnp.transpose(-probs, (0, 2, 1))
    iota = lax.broadcasted_iota(jnp.int32, (B, E, T), 2)
    neg_sorted, order = lax.sort((npt, iota), dimension=2, num_keys=1,
                                 is_stable=True)
    return _route_scan(neg_sorted, order, npt)


def _cast_body(a_ref, b_ref, c_ref, d_ref, ao_ref, bo_ref, co_ref, do_ref):
    ao_ref[...] = a_ref[...].astype(jnp.bfloat16)
    bo_ref[...] = b_ref[...].astype(jnp.bfloat16)
    co_ref[...] = c_ref[...].astype(jnp.bfloat16)
    do_ref[...] = d_ref[...].astype(jnp.bfloat16)


def _cast_weights(q_w, k_w, v_w, o_w):
    spec = pl.BlockSpec((D // 8, D), lambda i: (i, 0))
    out_sd = jax.ShapeDtypeStruct((D, D), jnp.bfloat16)
    return pl.pallas_call(
        _cast_body,
        grid=(8,),
        in_specs=[spec] * 4,
        out_specs=[spec] * 4,
        out_shape=[out_sd] * 4,
    )(q_w, k_w, v_w, o_w)


def _qkv_body(x_ref, qw_ref, kw_ref, vw_ref, qb_ref, kb_ref, vb_ref,
              lnw_ref, lnb_ref, q_ref, k_ref, v_ref):
    e = pl.program_id(1)
    xb = x_ref[0].astype(jnp.float32)
    mu = jnp.mean(xb, axis=1, keepdims=True)
    var = jnp.mean((xb - mu) ** 2, axis=1, keepdims=True)
    ln = (xb - mu) / jnp.sqrt(var + 1e-5) * lnw_ref[...] + lnb_ref[...]
    dn = (((1,), (1,)), ((), ()))
    for i in range(E):
        m = D >> i

        @pl.when(e == i)
        def _(m=m):
            ex = ln[:, :m].astype(jnp.bfloat16)
            q = jax.lax.dot_general(ex, qw_ref[:, :m], dn,
                                    preferred_element_type=jnp.float32) + qb_ref[...]
            k = jax.lax.dot_general(ex, kw_ref[:, :m], dn,
                                    preferred_element_type=jnp.float32) + kb_ref[...]
            v = jax.lax.dot_general(ex, vw_ref[:, :m], dn,
                                    preferred_element_type=jnp.float32) + vb_ref[...]
            q_ref[0] = q.astype(jnp.bfloat16)
            k_ref[0] = k.astype(jnp.bfloat16)
            v_ref[0] = v.astype(jnp.bfloat16)


def _qkv_all(xg_b, q_wb, k_wb, v_wb, q_b, k_b, v_b, ln_w, ln_b):
    xspec = pl.BlockSpec((1, N, D), lambda b, e: (b, e, 0))
    wspec = pl.BlockSpec((D, D), lambda b, e: (0, 0))
    bspec = pl.BlockSpec((D,), lambda b, e: (0,))
    ospec = pl.BlockSpec((1, N, D), lambda b, e: (b, e, 0))
    out_sd = jax.ShapeDtypeStruct((B, T, D), jnp.bfloat16)
    return pl.pallas_call(
        _qkv_body,
        grid=(B, E),
        in_specs=[xspec, wspec, wspec, wspec, bspec, bspec, bspec, bspec, bspec],
        out_specs=[ospec, ospec, ospec],
        out_shape=[out_sd, out_sd, out_sd],
    )(xg_b, q_wb, k_wb, v_wb, q_b, k_b, v_b, ln_w, ln_b)


def _attn_body(q_ref, k_ref, v_ref, o_ref):
    q = q_ref[0]
    k = k_ref[0]
    v = v_ref[0]
    s = jax.lax.dot_general(q, k, (((1,), (1,)), ((), ())),
                            preferred_element_type=jnp.float32) * SCALE
    p = jnp.exp(s)
    p = p / jnp.sum(p, axis=1, keepdims=True)
    o = jax.lax.dot_general(p.astype(jnp.bfloat16), v, (((1,), (0,)), ((), ())),
                            preferred_element_type=jnp.float32)
    o_ref[0] = o.astype(jnp.bfloat16)


def _attention(q, k, v, bq=512):
    qspec = pl.BlockSpec((1, bq, DH), lambda b, h, i: (b, i, h))
    kvspec = pl.BlockSpec((1, T, DH), lambda b, h, i: (b, 0, h))
    return pl.pallas_call(
        _attn_body,
        grid=(B, H, T // bq),
        in_specs=[qspec, kvspec, kvspec],
        out_specs=qspec,
        out_shape=jax.ShapeDtypeStruct((B, T, D), jnp.bfloat16),
    )(q, k, v)


def _oproj_body(a_ref, x_ref, ow_ref, ob_ref, o_ref):
    e = pl.program_id(1)
    ab = a_ref[0]
    xb = x_ref[0]
    dn = (((1,), (1,)), ((), ()))
    for i in range(E):
        m = D >> i

        @pl.when(e == i)
        def _(m=m):
            proj = jax.lax.dot_general(ab[:, :m], ow_ref[:m, :m], dn,
                                       preferred_element_type=jnp.float32) + ob_ref[:m]
            if m == D:
                o_ref[0] = xb + proj
            else:
                o_ref[0] = jnp.concatenate([xb[:, :m] + proj, xb[:, m:]], axis=1)


def _oproj_all(attn_g, x_g, o_wb, o_b):
    aspec = pl.BlockSpec((1, N, D), lambda b, e: (b, e, 0))
    wspec = pl.BlockSpec((D, D), lambda b, e: (0, 0))
    bspec = pl.BlockSpec((D,), lambda b, e: (0,))
    return pl.pallas_call(
        _oproj_body,
        grid=(B, E),
        in_specs=[aspec, aspec, wspec, bspec],
        out_specs=aspec,
        out_shape=jax.ShapeDtypeStruct((B, T, D), jnp.float32),
    )(attn_g, x_g, o_wb, o_b)


def kernel(x, router_prob, q_w, q_b, k_w, k_b, v_w, v_b, o_w, o_b, ln_w, ln_b):
    q_wb, k_wb, v_wb, o_wb = _cast_weights(q_w, k_w, v_w, o_w)

    perm = _greedy_route(router_prob)
    new_probs = jnp.take_along_axis(router_prob, perm[:, :, None], axis=1)
    xg = jnp.take_along_axis(x, perm[:, :, None], axis=1)

    q, k, v = _qkv_all(xg, q_wb, k_wb, v_wb, q_b, k_b, v_b, ln_w, ln_b)

    attn_out = _attention(q, k, v)

    perm2 = _greedy_route(new_probs)
    attn_g = jnp.take_along_axis(attn_out, perm2[:, :, None], axis=1)
    x_g = jnp.take_along_axis(x, perm2[:, :, None], axis=1)
    final_probs = jnp.take_along_axis(new_probs, perm2[:, :, None], axis=1)

    out = _oproj_all(attn_g, x_g, o_wb, o_b)
    return out, final_probs


# speedup vs baseline: 1.1356x; 1.0278x over previous
import functools
import jax
import jax.numpy as jnp
from jax import lax
from jax.experimental import pallas as pl
from jax.experimental.pallas import tpu as pltpu
from jax.experimental.pallas import tpu_sc as plsc

B = 2
T = 2048
D = 2048
E = 8
H = 8
N = T // E
DH = D // H
SCALE = D ** (-0.5)
L = 16


def _route_scan_body(vals_hbm, order_hbm, probs_hbm, perm_hbm,
                     vals_v, order_v, probs_v, claimed_v, perm_v):
    b = lax.axis_index("c")
    sid = lax.axis_index("s")

    @pl.when(sid == 0)
    def _():
        pltpu.sync_copy(vals_hbm.at[b], vals_v)
        pltpu.sync_copy(order_hbm.at[b], order_v)
        pltpu.sync_copy(probs_hbm.at[b], probs_v)
        zeros16 = jnp.zeros((L,), jnp.int32)
        ones16 = jnp.ones((L,), jnp.int32)
        n_vec = jnp.full((L,), N, jnp.int32)

        def zbody(g, c):
            claimed_v[pl.ds(pl.multiple_of(g * L, L), L)] = zeros16
            return c

        lax.fori_loop(0, T // L, zbody, 0)

        for e in range(E):
            def body(g, cnt, e=e):
                off = pl.multiple_of(g * L, L)
                idx16 = order_v[e, pl.ds(off, L)]
                v16 = vals_v[e, pl.ds(off, L)]
                av16 = plsc.load_gather(claimed_v, [idx16])
                pos = (av16 == 0) & (v16 < 0.0)
                c = plsc.cumsum(pos.astype(jnp.int32)) + cnt
                pick = pos & (c <= n_vec)
                plsc.store_scatter(claimed_v, [idx16], ones16, mask=pick)
                plsc.store_scatter(perm_v, [c + (e * N - 1)], idx16, mask=pick)
                npos = plsc.all_reduce_population_count(pos)
                return jnp.minimum(cnt + npos, n_vec)

            cnt = lax.fori_loop(0, T // L, body, jnp.zeros((L,), jnp.int32))

            @pl.when(jnp.max(cnt) < N)
            def _(e=e, cnt=cnt):
                need = n_vec - cnt

                def zt_body(g, cz, e=e, cnt=cnt, need=need):
                    off = pl.multiple_of(g * L, L)
                    cl16 = claimed_v[pl.ds(off, L)]
                    p16 = probs_v[e, pl.ds(off, L)]
                    zt = (cl16 != 0) | (p16 == 0.0)
                    c2 = plsc.cumsum(zt.astype(jnp.int32)) + cz
                    pickz = zt & (c2 <= need)
                    tok16 = lax.iota(jnp.int32, L) + off
                    plsc.store_scatter(claimed_v, [tok16], ones16, mask=pickz)
                    plsc.store_scatter(perm_v, [c2 + cnt + (e * N - 1)], tok16,
                                       mask=pickz)
                    nz = plsc.all_reduce_population_count(zt)
                    return jnp.minimum(cz + nz, need)

                lax.fori_loop(0, T // L, zt_body, jnp.zeros((L,), jnp.int32))

        pltpu.sync_copy(perm_v, perm_hbm.at[b])


_route_scan = functools.partial(
    pl.kernel,
    out_type=jax.ShapeDtypeStruct((B, T), jnp.int32),
    mesh=plsc.VectorSubcoreMesh(core_axis_name="c", subcore_axis_name="s"),
    compiler_params=pltpu.CompilerParams(needs_layout_passes=False),
    scratch_types=[
        pltpu.VMEM((E, T), jnp.float32),
        pltpu.VMEM((E, T), jnp.int32),
        pltpu.VMEM((E, T), jnp.float32),
        pltpu.VMEM((T,), jnp.int32),
        pltpu.VMEM((T,), jnp.int32),
    ],
)(_route_scan_body)


def _greedy_route(probs):
    npt = jnp.transpose(-probs, (0, 2, 1))
    iota = lax.broadcasted_iota(jnp.int32, (B, E, T), 2)
    neg_sorted, order = lax.sort((npt, iota), dimension=2, num_keys=1,
                                 is_stable=True)
    return _route_scan(neg_sorted, order, npt)


def _cast_body(a_ref, b_ref, c_ref, d_ref, ao_ref, bo_ref, co_ref, do_ref):
    ao_ref[...] = a_ref[...].astype(jnp.bfloat16)
    bo_ref[...] = b_ref[...].astype(jnp.bfloat16)
    co_ref[...] = c_ref[...].astype(jnp.bfloat16)
    do_ref[...] = d_ref[...].astype(jnp.bfloat16)


def _cast_weights(q_w, k_w, v_w, o_w):
    spec = pl.BlockSpec((D // 8, D), lambda i: (i, 0))
    out_sd = jax.ShapeDtypeStruct((D, D), jnp.bfloat16)
    return pl.pallas_call(
        _cast_body,
        grid=(8,),
        in_specs=[spec] * 4,
        out_specs=[spec] * 4,
        out_shape=[out_sd] * 4,
    )(q_w, k_w, v_w, o_w)


def _qkv_body(x_ref, qw_ref, kw_ref, vw_ref, qb_ref, kb_ref, vb_ref,
              lnw_ref, lnb_ref, q_ref, k_ref, v_ref):
    e = pl.program_id(1)
    xb = x_ref[0].astype(jnp.float32)
    mu = jnp.mean(xb, axis=1, keepdims=True)
    var = jnp.mean((xb - mu) ** 2, axis=1, keepdims=True)
    ln = (xb - mu) / jnp.sqrt(var + 1e-5) * lnw_ref[...] + lnb_ref[...]
    dn = (((1,), (1,)), ((), ()))
    for i in range(E):
        m = D >> i

        @pl.when(e == i)
        def _(m=m):
            ex = ln[:, :m].astype(jnp.bfloat16)
            q = jax.lax.dot_general(ex, qw_ref[:, :m], dn,
                                    preferred_element_type=jnp.float32) + qb_ref[...]
            k = jax.lax.dot_general(ex, kw_ref[:, :m], dn,
                                    preferred_element_type=jnp.float32) + kb_ref[...]
            v = jax.lax.dot_general(ex, vw_ref[:, :m], dn,
                                    preferred_element_type=jnp.float32) + vb_ref[...]
            q_ref[0] = q.astype(jnp.bfloat16)
            k_ref[0] = k.astype(jnp.bfloat16)
            v_ref[0] = v.astype(jnp.bfloat16)


def _qkv_all(xg_b, q_wb, k_wb, v_wb, q_b, k_b, v_b, ln_w, ln_b):
    xspec = pl.BlockSpec((1, N, D), lambda b, e: (b, e, 0))
    wspec = pl.BlockSpec((D, D), lambda b, e: (0, 0))
    bspec = pl.BlockSpec((D,), lambda b, e: (0,))
    ospec = pl.BlockSpec((1, N, D), lambda b, e: (b, e, 0))
    out_sd = jax.ShapeDtypeStruct((B, T, D), jnp.bfloat16)
    return pl.pallas_call(
        _qkv_body,
        grid=(B, E),
        in_specs=[xspec, wspec, wspec, wspec, bspec, bspec, bspec, bspec, bspec],
        out_specs=[ospec, ospec, ospec],
        out_shape=[out_sd, out_sd, out_sd],
    )(xg_b, q_wb, k_wb, v_wb, q_b, k_b, v_b, ln_w, ln_b)


def _attn_body(q_ref, k_ref, v_ref, o_ref):
    q = q_ref[0]
    k = k_ref[0]
    v = v_ref[0]
    s = jax.lax.dot_general(q, k, (((1,), (1,)), ((), ())),
                            preferred_element_type=jnp.float32) * SCALE
    p = jnp.exp(s)
    p = p / jnp.sum(p, axis=1, keepdims=True)
    o = jax.lax.dot_general(p.astype(jnp.bfloat16), v, (((1,), (0,)), ((), ())),
                            preferred_element_type=jnp.float32)
    o_ref[0] = o.astype(jnp.bfloat16)


def _attention(q, k, v, bq=1024):
    qspec = pl.BlockSpec((1, bq, DH), lambda b, h, i: (b, i, h))
    kvspec = pl.BlockSpec((1, T, DH), lambda b, h, i: (b, 0, h))
    return pl.pallas_call(
        _attn_body,
        grid=(B, H, T // bq),
        in_specs=[qspec, kvspec, kvspec],
        out_specs=qspec,
        out_shape=jax.ShapeDtypeStruct((B, T, D), jnp.bfloat16),
    )(q, k, v)


def _oproj_body(a_ref, x_ref, ow_ref, ob_ref, o_ref):
    e = pl.program_id(1)
    ab = a_ref[0]
    xb = x_ref[0]
    dn = (((1,), (1,)), ((), ()))
    for i in range(E):
        m = D >> i

        @pl.when(e == i)
        def _(m=m):
            proj = jax.lax.dot_general(ab[:, :m], ow_ref[:m, :m], dn,
                                       preferred_element_type=jnp.float32) + ob_ref[:m]
            if m == D:
                o_ref[0] = xb + proj
            else:
                o_ref[0] = jnp.concatenate([xb[:, :m] + proj, xb[:, m:]], axis=1)


def _oproj_all(attn_g, x_g, o_wb, o_b):
    aspec = pl.BlockSpec((1, N, D), lambda b, e: (b, e, 0))
    wspec = pl.BlockSpec((D, D), lambda b, e: (0, 0))
    bspec = pl.BlockSpec((D,), lambda b, e: (0,))
    return pl.pallas_call(
        _oproj_body,
        grid=(B, E),
        in_specs=[aspec, aspec, wspec, bspec],
        out_specs=aspec,
        out_shape=jax.ShapeDtypeStruct((B, T, D), jnp.float32),
    )(attn_g, x_g, o_wb, o_b)


def kernel(x, router_prob, q_w, q_b, k_w, k_b, v_w, v_b, o_w, o_b, ln_w, ln_b):
    q_wb, k_wb, v_wb, o_wb = _cast_weights(q_w, k_w, v_w, o_w)

    perm = _greedy_route(router_prob)
    new_probs = jnp.take_along_axis(router_prob, perm[:, :, None], axis=1)
    xg = jnp.take_along_axis(x, perm[:, :, None], axis=1)

    q, k, v = _qkv_all(xg, q_wb, k_wb, v_wb, q_b, k_b, v_b, ln_w, ln_b)

    attn_out = _attention(q, k, v)

    perm2 = _greedy_route(new_probs)
    attn_g = jnp.take_along_axis(attn_out, perm2[:, :, None], axis=1)
    x_g = jnp.take_along_axis(x, perm2[:, :, None], axis=1)
    final_probs = jnp.take_along_axis(new_probs, perm2[:, :, None], axis=1)

    out = _oproj_all(attn_g, x_g, o_wb, o_b)
    return out, final_probs


# speedup vs baseline: 1.1513x; 1.0138x over previous
import functools
import jax
import jax.numpy as jnp
from jax import lax
from jax.experimental import pallas as pl
from jax.experimental.pallas import tpu as pltpu
from jax.experimental.pallas import tpu_sc as plsc

B = 2
T = 2048
D = 2048
E = 8
H = 8
N = T // E
DH = D // H
SCALE = D ** (-0.5)
L = 16


def _route_scan_body(vals_hbm, order_hbm, probs_hbm, perm_hbm,
                     vals_v, order_v, probs_v, claimed_v, perm_v):
    b = lax.axis_index("c")
    sid = lax.axis_index("s")

    @pl.when(sid == 0)
    def _():
        pltpu.sync_copy(vals_hbm.at[b], vals_v)
        pltpu.sync_copy(order_hbm.at[b], order_v)
        pltpu.sync_copy(probs_hbm.at[b], probs_v)
        zeros16 = jnp.zeros((L,), jnp.int32)
        ones16 = jnp.ones((L,), jnp.int32)
        n_vec = jnp.full((L,), N, jnp.int32)

        def zbody(g, c):
            claimed_v[pl.ds(pl.multiple_of(g * L, L), L)] = zeros16
            return c

        lax.fori_loop(0, T // L, zbody, 0)

        for e in range(E):
            def body(g, cnt, e=e):
                off = pl.multiple_of(g * L, L)
                idx16 = order_v[e, pl.ds(off, L)]
                v16 = vals_v[e, pl.ds(off, L)]
                av16 = plsc.load_gather(claimed_v, [idx16])
                pos = (av16 == 0) & (v16 < 0.0)
                c = plsc.cumsum(pos.astype(jnp.int32)) + cnt
                pick = pos & (c <= n_vec)
                plsc.store_scatter(claimed_v, [idx16], ones16, mask=pick)
                plsc.store_scatter(perm_v, [c + (e * N - 1)], idx16, mask=pick)
                npos = plsc.all_reduce_population_count(pos)
                return jnp.minimum(cnt + npos, n_vec)

            cnt = lax.fori_loop(0, T // L, body, jnp.zeros((L,), jnp.int32))

            @pl.when(jnp.max(cnt) < N)
            def _(e=e, cnt=cnt):
                need = n_vec - cnt

                def zt_body(g, cz, e=e, cnt=cnt, need=need):
                    off = pl.multiple_of(g * L, L)
                    cl16 = claimed_v[pl.ds(off, L)]
                    p16 = probs_v[e, pl.ds(off, L)]
                    zt = (cl16 != 0) | (p16 == 0.0)
                    c2 = plsc.cumsum(zt.astype(jnp.int32)) + cz
                    pickz = zt & (c2 <= need)
                    tok16 = lax.iota(jnp.int32, L) + off
                    plsc.store_scatter(claimed_v, [tok16], ones16, mask=pickz)
                    plsc.store_scatter(perm_v, [c2 + cnt + (e * N - 1)], tok16,
                                       mask=pickz)
                    nz = plsc.all_reduce_population_count(zt)
                    return jnp.minimum(cz + nz, need)

                lax.fori_loop(0, T // L, zt_body, jnp.zeros((L,), jnp.int32))

        pltpu.sync_copy(perm_v, perm_hbm.at[b])


_route_scan = functools.partial(
    pl.kernel,
    out_type=jax.ShapeDtypeStruct((B, T), jnp.int32),
    mesh=plsc.VectorSubcoreMesh(core_axis_name="c", subcore_axis_name="s"),
    compiler_params=pltpu.CompilerParams(needs_layout_passes=False),
    scratch_types=[
        pltpu.VMEM((E, T), jnp.float32),
        pltpu.VMEM((E, T), jnp.int32),
        pltpu.VMEM((E, T), jnp.float32),
        pltpu.VMEM((T,), jnp.int32),
        pltpu.VMEM((T,), jnp.int32),
    ],
)(_route_scan_body)


def _greedy_route(probs):
    npt = jnp.transpose(-probs, (0, 2, 1))
    iota = lax.broadcasted_iota(jnp.int32, (B, E, T), 2)
    neg_sorted, order = lax.sort((npt, iota), dimension=2, num_keys=1,
                                 is_stable=True)
    return _route_scan(neg_sorted, order, npt)


def _cast_body(a_ref, b_ref, c_ref, d_ref, ao_ref, bo_ref, co_ref, do_ref):
    ao_ref[...] = a_ref[...].astype(jnp.bfloat16)
    bo_ref[...] = b_ref[...].astype(jnp.bfloat16)
    co_ref[...] = c_ref[...].astype(jnp.bfloat16)
    do_ref[...] = d_ref[...].astype(jnp.bfloat16)


def _cast_weights(q_w, k_w, v_w, o_w):
    spec = pl.BlockSpec((D // 8, D), lambda i: (i, 0))
    out_sd = jax.ShapeDtypeStruct((D, D), jnp.bfloat16)
    return pl.pallas_call(
        _cast_body,
        grid=(8,),
        in_specs=[spec] * 4,
        out_specs=[spec] * 4,
        out_shape=[out_sd] * 4,
    )(q_w, k_w, v_w, o_w)


def _qkv_body(x_ref, qw_ref, kw_ref, vw_ref, qb_ref, kb_ref, vb_ref,
              lnw_ref, lnb_ref, q_ref, k_ref, v_ref):
    e = pl.program_id(1)
    xb = x_ref[0].astype(jnp.float32)
    mu = jnp.mean(xb, axis=1, keepdims=True)
    var = jnp.mean((xb - mu) ** 2, axis=1, keepdims=True)
    ln = (xb - mu) / jnp.sqrt(var + 1e-5) * lnw_ref[...] + lnb_ref[...]
    dn = (((1,), (1,)), ((), ()))
    for i in range(E):
        m = D >> i

        @pl.when(e == i)
        def _(m=m):
            ex = ln[:, :m].astype(jnp.bfloat16)
            q = jax.lax.dot_general(ex, qw_ref[:, :m], dn,
                                    preferred_element_type=jnp.float32) + qb_ref[...]
            k = jax.lax.dot_general(ex, kw_ref[:, :m], dn,
                                    preferred_element_type=jnp.float32) + kb_ref[...]
            v = jax.lax.dot_general(ex, vw_ref[:, :m], dn,
                                    preferred_element_type=jnp.float32) + vb_ref[...]
            q_ref[0] = q.astype(jnp.bfloat16)
            k_ref[0] = k.astype(jnp.bfloat16)
            v_ref[0] = v.astype(jnp.bfloat16)


def _qkv_all(xg_b, q_wb, k_wb, v_wb, q_b, k_b, v_b, ln_w, ln_b):
    xspec = pl.BlockSpec((1, N, D), lambda b, e: (b, e, 0))
    wspec = pl.BlockSpec((D, D), lambda b, e: (0, 0))
    bspec = pl.BlockSpec((D,), lambda b, e: (0,))
    ospec = pl.BlockSpec((1, N, D), lambda b, e: (b, e, 0))
    out_sd = jax.ShapeDtypeStruct((B, T, D), jnp.bfloat16)
    return pl.pallas_call(
        _qkv_body,
        grid=(B, E),
        in_specs=[xspec, wspec, wspec, wspec, bspec, bspec, bspec, bspec, bspec],
        out_specs=[ospec, ospec, ospec],
        out_shape=[out_sd, out_sd, out_sd],
    )(xg_b, q_wb, k_wb, v_wb, q_b, k_b, v_b, ln_w, ln_b)


def _attn_body(q_ref, k_ref, v_ref, o_ref):
    q = q_ref[0]
    k = k_ref[0]
    v = v_ref[0]
    s = jax.lax.dot_general(q, k, (((1,), (1,)), ((), ())),
                            preferred_element_type=jnp.float32) * SCALE
    p = jnp.exp(s)
    p = p / jnp.sum(p, axis=1, keepdims=True)
    o = jax.lax.dot_general(p.astype(jnp.bfloat16), v, (((1,), (0,)), ((), ())),
                            preferred_element_type=jnp.float32)
    o_ref[0] = o.astype(jnp.bfloat16)


def _attention(q, k, v, bq=2048):
    qspec = pl.BlockSpec((1, bq, DH), lambda b, h, i: (b, i, h))
    kvspec = pl.BlockSpec((1, T, DH), lambda b, h, i: (b, 0, h))
    return pl.pallas_call(
        _attn_body,
        grid=(B, H, T // bq),
        in_specs=[qspec, kvspec, kvspec],
        out_specs=qspec,
        out_shape=jax.ShapeDtypeStruct((B, T, D), jnp.bfloat16),
    )(q, k, v)


def _oproj_body(a_ref, x_ref, ow_ref, ob_ref, o_ref):
    e = pl.program_id(1)
    ab = a_ref[0]
    xb = x_ref[0]
    dn = (((1,), (1,)), ((), ()))
    for i in range(E):
        m = D >> i

        @pl.when(e == i)
        def _(m=m):
            proj = jax.lax.dot_general(ab[:, :m], ow_ref[:m, :m], dn,
                                       preferred_element_type=jnp.float32) + ob_ref[:m]
            if m == D:
                o_ref[0] = xb + proj
            else:
                o_ref[0] = jnp.concatenate([xb[:, :m] + proj, xb[:, m:]], axis=1)


def _oproj_all(attn_g, x_g, o_wb, o_b):
    aspec = pl.BlockSpec((1, N, D), lambda b, e: (b, e, 0))
    wspec = pl.BlockSpec((D, D), lambda b, e: (0, 0))
    bspec = pl.BlockSpec((D,), lambda b, e: (0,))
    return pl.pallas_call(
        _oproj_body,
        grid=(B, E),
        in_specs=[aspec, aspec, wspec, bspec],
        out_specs=aspec,
        out_shape=jax.ShapeDtypeStruct((B, T, D), jnp.float32),
    )(attn_g, x_g, o_wb, o_b)


def kernel(x, router_prob, q_w, q_b, k_w, k_b, v_w, v_b, o_w, o_b, ln_w, ln_b):
    q_wb, k_wb, v_wb, o_wb = _cast_weights(q_w, k_w, v_w, o_w)

    perm = _greedy_route(router_prob)
    new_probs = jnp.take_along_axis(router_prob, perm[:, :, None], axis=1)
    xg = jnp.take_along_axis(x, perm[:, :, None], axis=1)

    q, k, v = _qkv_all(xg, q_wb, k_wb, v_wb, q_b, k_b, v_b, ln_w, ln_b)

    attn_out = _attention(q, k, v)

    perm2 = _greedy_route(new_probs)
    attn_g = jnp.take_along_axis(attn_out, perm2[:, :, None], axis=1)
    x_g = jnp.take_along_axis(x, perm2[:, :, None], axis=1)
    final_probs = jnp.take_along_axis(new_probs, perm2[:, :, None], axis=1)

    out = _oproj_all(attn_g, x_g, o_wb, o_b)
    return out, final_probs


# speedup vs baseline: 1.1918x; 1.0352x over previous
import functools
import jax
import jax.numpy as jnp
from jax import lax
from jax.experimental import pallas as pl
from jax.experimental.pallas import tpu as pltpu
from jax.experimental.pallas import tpu_sc as plsc

B = 2
T = 2048
D = 2048
E = 8
H = 8
N = T // E
DH = D // H
SCALE = D ** (-0.5)
L = 16


def _route_scan_body(vals_hbm, order_hbm, probs_hbm, perm_hbm, np_hbm,
                     vals_v, order_v, probs_v, claimed_v, perm_v, np_v,
                     *, negate_out):
    b = lax.axis_index("c")
    sid = lax.axis_index("s")

    @pl.when(sid == 0)
    def _():
        pltpu.sync_copy(vals_hbm.at[b], vals_v)
        pltpu.sync_copy(order_hbm.at[b], order_v)
        pltpu.sync_copy(probs_hbm.at[b], probs_v)
        zeros16 = jnp.zeros((L,), jnp.int32)
        ones16 = jnp.ones((L,), jnp.int32)
        n_vec = jnp.full((L,), N, jnp.int32)

        def zbody(g, c):
            claimed_v[pl.ds(pl.multiple_of(g * L, L), L)] = zeros16
            return c

        lax.fori_loop(0, T // L, zbody, 0)

        for e in range(E):
            def body(g, cnt, e=e):
                off = pl.multiple_of(g * L, L)
                idx16 = order_v[e, pl.ds(off, L)]
                v16 = vals_v[e, pl.ds(off, L)]
                av16 = plsc.load_gather(claimed_v, [idx16])
                pos = (av16 == 0) & (v16 < 0.0)
                c = plsc.cumsum(pos.astype(jnp.int32)) + cnt
                pick = pos & (c <= n_vec)
                plsc.store_scatter(claimed_v, [idx16], ones16, mask=pick)
                plsc.store_scatter(perm_v, [c + (e * N - 1)], idx16, mask=pick)
                npos = plsc.all_reduce_population_count(pos)
                return jnp.minimum(cnt + npos, n_vec)

            cnt = lax.fori_loop(0, T // L, body, jnp.zeros((L,), jnp.int32))

            @pl.when(jnp.max(cnt) < N)
            def _(e=e, cnt=cnt):
                need = n_vec - cnt

                def zt_body(g, cz, e=e, cnt=cnt, need=need):
                    off = pl.multiple_of(g * L, L)
                    cl16 = claimed_v[pl.ds(off, L)]
                    p16 = probs_v[e, pl.ds(off, L)]
                    zt = (cl16 != 0) | (p16 == 0.0)
                    c2 = plsc.cumsum(zt.astype(jnp.int32)) + cz
                    pickz = zt & (c2 <= need)
                    tok16 = lax.iota(jnp.int32, L) + off
                    plsc.store_scatter(claimed_v, [tok16], ones16, mask=pickz)
                    plsc.store_scatter(perm_v, [c2 + cnt + (e * N - 1)], tok16,
                                       mask=pickz)
                    nz = plsc.all_reduce_population_count(zt)
                    return jnp.minimum(cz + nz, need)

                lax.fori_loop(0, T // L, zt_body, jnp.zeros((L,), jnp.int32))

        def gbody(g, c, negate_out=negate_out):
            off = pl.multiple_of(g * L, L)
            j16 = perm_v[pl.ds(off, L)]
            for e in range(E):
                ev = jnp.full((L,), e, jnp.int32)
                v = plsc.load_gather(probs_v, [ev, j16])
                if negate_out:
                    v = jnp.zeros((L,), jnp.float32) - v
                np_v[e, pl.ds(off, L)] = v
            return c

        lax.fori_loop(0, T // L, gbody, 0)
        pltpu.sync_copy(perm_v, perm_hbm.at[b])
        pltpu.sync_copy(np_v, np_hbm.at[b])


def _make_route_scan(negate_out):
    return functools.partial(
        pl.kernel,
        out_type=(jax.ShapeDtypeStruct((B, T), jnp.int32),
                  jax.ShapeDtypeStruct((B, E, T), jnp.float32)),
        mesh=plsc.VectorSubcoreMesh(core_axis_name="c", subcore_axis_name="s"),
        compiler_params=pltpu.CompilerParams(needs_layout_passes=False),
        scratch_types=[
            pltpu.VMEM((E, T), jnp.float32),
            pltpu.VMEM((E, T), jnp.int32),
            pltpu.VMEM((E, T), jnp.float32),
            pltpu.VMEM((T,), jnp.int32),
            pltpu.VMEM((T,), jnp.int32),
            pltpu.VMEM((E, T), jnp.float32),
        ],
    )(functools.partial(_route_scan_body, negate_out=negate_out))


_route_scan_keep = _make_route_scan(False)
_route_scan_neg = _make_route_scan(True)


def _greedy_route(neg_probs_t, negate_out):
    iota = lax.broadcasted_iota(jnp.int32, (B, E, T), 2)
    neg_sorted, order = lax.sort((neg_probs_t, iota), dimension=2, num_keys=1,
                                 is_stable=True)
    scan = _route_scan_neg if negate_out else _route_scan_keep
    return scan(neg_sorted, order, neg_probs_t)


def _cast_body(a_ref, b_ref, c_ref, d_ref, ao_ref, bo_ref, co_ref, do_ref):
    ao_ref[...] = a_ref[...].astype(jnp.bfloat16)
    bo_ref[...] = b_ref[...].astype(jnp.bfloat16)
    co_ref[...] = c_ref[...].astype(jnp.bfloat16)
    do_ref[...] = d_ref[...].astype(jnp.bfloat16)


def _cast_weights(q_w, k_w, v_w, o_w):
    spec = pl.BlockSpec((D // 8, D), lambda i: (i, 0))
    out_sd = jax.ShapeDtypeStruct((D, D), jnp.bfloat16)
    return pl.pallas_call(
        _cast_body,
        grid=(8,),
        in_specs=[spec] * 4,
        out_specs=[spec] * 4,
        out_shape=[out_sd] * 4,
    )(q_w, k_w, v_w, o_w)


def _qkv_body(x_ref, qw_ref, kw_ref, vw_ref, qb_ref, kb_ref, vb_ref,
              lnw_ref, lnb_ref, q_ref, k_ref, v_ref):
    e = pl.program_id(1)
    xb = x_ref[0].astype(jnp.float32)
    mu = jnp.mean(xb, axis=1, keepdims=True)
    var = jnp.mean((xb - mu) ** 2, axis=1, keepdims=True)
    ln = (xb - mu) / jnp.sqrt(var + 1e-5) * lnw_ref[...] + lnb_ref[...]
    dn = (((1,), (1,)), ((), ()))
    for i in range(E):
        m = D >> i

        @pl.when(e == i)
        def _(m=m):
            ex = ln[:, :m].astype(jnp.bfloat16)
            q = jax.lax.dot_general(ex, qw_ref[:, :m], dn,
                                    preferred_element_type=jnp.float32) + qb_ref[...]
            k = jax.lax.dot_general(ex, kw_ref[:, :m], dn,
                                    preferred_element_type=jnp.float32) + kb_ref[...]
            v = jax.lax.dot_general(ex, vw_ref[:, :m], dn,
                                    preferred_element_type=jnp.float32) + vb_ref[...]
            q_ref[0] = q.astype(jnp.bfloat16)
            k_ref[0] = k.astype(jnp.bfloat16)
            v_ref[0] = v.astype(jnp.bfloat16)


def _qkv_all(xg_b, q_wb, k_wb, v_wb, q_b, k_b, v_b, ln_w, ln_b):
    xspec = pl.BlockSpec((1, N, D), lambda b, e: (b, e, 0))
    wspec = pl.BlockSpec((D, D), lambda b, e: (0, 0))
    bspec = pl.BlockSpec((D,), lambda b, e: (0,))
    ospec = pl.BlockSpec((1, N, D), lambda b, e: (b, e, 0))
    out_sd = jax.ShapeDtypeStruct((B, T, D), jnp.bfloat16)
    return pl.pallas_call(
        _qkv_body,
        grid=(B, E),
        in_specs=[xspec, wspec, wspec, wspec, bspec, bspec, bspec, bspec, bspec],
        out_specs=[ospec, ospec, ospec],
        out_shape=[out_sd, out_sd, out_sd],
    )(xg_b, q_wb, k_wb, v_wb, q_b, k_b, v_b, ln_w, ln_b)


def _attn_body(q_ref, k_ref, v_ref, o_ref):
    q = q_ref[0]
    k = k_ref[0]
    v = v_ref[0]
    s = jax.lax.dot_general(q, k, (((1,), (1,)), ((), ())),
                            preferred_element_type=jnp.float32) * SCALE
    p = jnp.exp(s)
    p = p / jnp.sum(p, axis=1, keepdims=True)
    o = jax.lax.dot_general(p.astype(jnp.bfloat16), v, (((1,), (0,)), ((), ())),
                            preferred_element_type=jnp.float32)
    o_ref[0] = o.astype(jnp.bfloat16)


def _attention(q, k, v, bq=2048):
    qspec = pl.BlockSpec((1, bq, DH), lambda b, h, i: (b, i, h))
    kvspec = pl.BlockSpec((1, T, DH), lambda b, h, i: (b, 0, h))
    return pl.pallas_call(
        _attn_body,
        grid=(B, H, T // bq),
        in_specs=[qspec, kvspec, kvspec],
        out_specs=qspec,
        out_shape=jax.ShapeDtypeStruct((B, T, D), jnp.bfloat16),
    )(q, k, v)


def _oproj_body(a_ref, x_ref, ow_ref, ob_ref, o_ref):
    e = pl.program_id(1)
    ab = a_ref[0]
    xb = x_ref[0]
    dn = (((1,), (1,)), ((), ()))
    for i in range(E):
        m = D >> i

        @pl.when(e == i)
        def _(m=m):
            proj = jax.lax.dot_general(ab[:, :m], ow_ref[:m, :m], dn,
                                       preferred_element_type=jnp.float32) + ob_ref[:m]
            if m == D:
                o_ref[0] = xb + proj
            else:
                o_ref[0] = jnp.concatenate([xb[:, :m] + proj, xb[:, m:]], axis=1)


def _oproj_all(attn_g, x_g, o_wb, o_b):
    aspec = pl.BlockSpec((1, N, D), lambda b, e: (b, e, 0))
    wspec = pl.BlockSpec((D, D), lambda b, e: (0, 0))
    bspec = pl.BlockSpec((D,), lambda b, e: (0,))
    return pl.pallas_call(
        _oproj_body,
        grid=(B, E),
        in_specs=[aspec, aspec, wspec, bspec],
        out_specs=aspec,
        out_shape=jax.ShapeDtypeStruct((B, T, D), jnp.float32),
    )(attn_g, x_g, o_wb, o_b)


def kernel(x, router_prob, q_w, q_b, k_w, k_b, v_w, v_b, o_w, o_b, ln_w, ln_b):
    q_wb, k_wb, v_wb, o_wb = _cast_weights(q_w, k_w, v_w, o_w)

    npt = jnp.transpose(-router_prob, (0, 2, 1))
    perm, nnp_t = _greedy_route(npt, negate_out=False)
    xg = jnp.take_along_axis(x, perm[:, :, None], axis=1)

    q, k, v = _qkv_all(xg, q_wb, k_wb, v_wb, q_b, k_b, v_b, ln_w, ln_b)

    attn_out = _attention(q, k, v)

    perm2, fp_t = _greedy_route(nnp_t, negate_out=True)
    attn_g = jnp.take_along_axis(attn_out, perm2[:, :, None], axis=1)
    x_g = jnp.take_along_axis(x, perm2[:, :, None], axis=1)
    final_probs = jnp.transpose(fp_t, (0, 2, 1))

    out = _oproj_all(attn_g, x_g, o_wb, o_b)
    return out, final_probs


# speedup vs baseline: 1.2117x; 1.0167x over previous
import functools
import jax
import jax.numpy as jnp
from jax import lax
from jax.experimental import pallas as pl
from jax.experimental.pallas import tpu as pltpu
from jax.experimental.pallas import tpu_sc as plsc

B = 2
T = 2048
D = 2048
E = 8
H = 8
N = T // E
DH = D // H
SCALE = D ** (-0.5)
L = 16


def _route_scan_body(vals_hbm, order_hbm, probs_hbm, perm_hbm, np_hbm,
                     vals_v, order_v, probs_v, claimed_v, perm_v, np_v,
                     *, negate_out):
    b = lax.axis_index("c")
    sid = lax.axis_index("s")

    @pl.when(sid == 0)
    def _():
        pltpu.sync_copy(vals_hbm.at[b], vals_v)
        pltpu.sync_copy(order_hbm.at[b], order_v)
        pltpu.sync_copy(probs_hbm.at[b], probs_v)
        zeros16 = jnp.zeros((L,), jnp.int32)
        ones16 = jnp.ones((L,), jnp.int32)
        n_vec = jnp.full((L,), N, jnp.int32)

        def zbody(g, c):
            claimed_v[pl.ds(pl.multiple_of(g * L, L), L)] = zeros16
            return c

        lax.fori_loop(0, T // L, zbody, 0)

        for e in range(E):
            def body(g, cnt, e=e):
                off = pl.multiple_of(g * L, L)
                idx16 = order_v[e, pl.ds(off, L)]
                v16 = vals_v[e, pl.ds(off, L)]
                av16 = plsc.load_gather(claimed_v, [idx16])
                pos = (av16 == 0) & (v16 < 0.0)
                c = plsc.cumsum(pos.astype(jnp.int32)) + cnt
                pick = pos & (c <= n_vec)
                plsc.store_scatter(claimed_v, [idx16], ones16, mask=pick)
                plsc.store_scatter(perm_v, [c + (e * N - 1)], idx16, mask=pick)
                npos = plsc.all_reduce_population_count(pos)
                return jnp.minimum(cnt + npos, n_vec)

            cnt = lax.fori_loop(0, T // L, body, jnp.zeros((L,), jnp.int32))

            @pl.when(jnp.max(cnt) < N)
            def _(e=e, cnt=cnt):
                need = n_vec - cnt

                def zt_body(g, cz, e=e, cnt=cnt, need=need):
                    off = pl.multiple_of(g * L, L)
                    cl16 = claimed_v[pl.ds(off, L)]
                    p16 = probs_v[e, pl.ds(off, L)]
                    zt = (cl16 != 0) | (p16 == 0.0)
                    c2 = plsc.cumsum(zt.astype(jnp.int32)) + cz
                    pickz = zt & (c2 <= need)
                    tok16 = lax.iota(jnp.int32, L) + off
                    plsc.store_scatter(claimed_v, [tok16], ones16, mask=pickz)
                    plsc.store_scatter(perm_v, [c2 + cnt + (e * N - 1)], tok16,
                                       mask=pickz)
                    nz = plsc.all_reduce_population_count(zt)
                    return jnp.minimum(cz + nz, need)

                lax.fori_loop(0, T // L, zt_body, jnp.zeros((L,), jnp.int32))

        def gbody(g, c, negate_out=negate_out):
            off = pl.multiple_of(g * L, L)
            j16 = perm_v[pl.ds(off, L)]
            for e in range(E):
                ev = jnp.full((L,), e, jnp.int32)
                v = plsc.load_gather(probs_v, [ev, j16])
                if negate_out:
                    v = jnp.zeros((L,), jnp.float32) - v
                np_v[e, pl.ds(off, L)] = v
            return c

        lax.fori_loop(0, T // L, gbody, 0)
        pltpu.sync_copy(perm_v, perm_hbm.at[b])
        pltpu.sync_copy(np_v, np_hbm.at[b])


def _make_route_scan(negate_out):
    return functools.partial(
        pl.kernel,
        out_type=(jax.ShapeDtypeStruct((B, T), jnp.int32),
                  jax.ShapeDtypeStruct((B, E, T), jnp.float32)),
        mesh=plsc.VectorSubcoreMesh(core_axis_name="c", subcore_axis_name="s"),
        compiler_params=pltpu.CompilerParams(needs_layout_passes=False),
        scratch_types=[
            pltpu.VMEM((E, T), jnp.float32),
            pltpu.VMEM((E, T), jnp.int32),
            pltpu.VMEM((E, T), jnp.float32),
            pltpu.VMEM((T,), jnp.int32),
            pltpu.VMEM((T,), jnp.int32),
            pltpu.VMEM((E, T), jnp.float32),
        ],
    )(functools.partial(_route_scan_body, negate_out=negate_out))


_route_scan_keep = _make_route_scan(False)
_route_scan_neg = _make_route_scan(True)


def _greedy_route(neg_probs_t, negate_out):
    iota = lax.broadcasted_iota(jnp.int32, (B, E, T), 2)
    neg_sorted, order = lax.sort((neg_probs_t, iota), dimension=2, num_keys=1,
                                 is_stable=True)
    scan = _route_scan_neg if negate_out else _route_scan_keep
    return scan(neg_sorted, order, neg_probs_t)


def _cast_body(a_ref, b_ref, c_ref, d_ref, ao_ref, bo_ref, co_ref, do_ref):
    ao_ref[...] = a_ref[...].astype(jnp.bfloat16)
    bo_ref[...] = b_ref[...].astype(jnp.bfloat16)
    co_ref[...] = c_ref[...].astype(jnp.bfloat16)
    do_ref[...] = d_ref[...].astype(jnp.bfloat16)


def _cast_weights(q_w, k_w, v_w, o_w):
    spec = pl.BlockSpec((D // 8, D), lambda i: (i, 0))
    out_sd = jax.ShapeDtypeStruct((D, D), jnp.bfloat16)
    return pl.pallas_call(
        _cast_body,
        grid=(8,),
        in_specs=[spec] * 4,
        out_specs=[spec] * 4,
        out_shape=[out_sd] * 4,
    )(q_w, k_w, v_w, o_w)


def _qkv_body(x_ref, qw_ref, kw_ref, vw_ref, qb_ref, kb_ref, vb_ref,
              lnw_ref, lnb_ref, q_ref, k_ref, v_ref):
    e = pl.program_id(1)
    xb = x_ref[0].astype(jnp.float32)
    mu = jnp.mean(xb, axis=1, keepdims=True)
    var = jnp.mean((xb - mu) ** 2, axis=1, keepdims=True)
    ln = (xb - mu) / jnp.sqrt(var + 1e-5) * lnw_ref[...] + lnb_ref[...]
    dn = (((1,), (1,)), ((), ()))
    for i in range(E):
        m = D >> i

        @pl.when(e == i)
        def _(m=m):
            ex = ln[:, :m].astype(jnp.bfloat16)
            q = jax.lax.dot_general(ex, qw_ref[:, :m], dn,
                                    preferred_element_type=jnp.float32) + qb_ref[...]
            k = jax.lax.dot_general(ex, kw_ref[:, :m], dn,
                                    preferred_element_type=jnp.float32) + kb_ref[...]
            v = jax.lax.dot_general(ex, vw_ref[:, :m], dn,
                                    preferred_element_type=jnp.float32) + vb_ref[...]
            q_ref[0] = (q * SCALE).astype(jnp.bfloat16)
            k_ref[0] = k.astype(jnp.bfloat16)
            v_ref[0] = v.astype(jnp.bfloat16)


def _qkv_all(xg_b, q_wb, k_wb, v_wb, q_b, k_b, v_b, ln_w, ln_b):
    xspec = pl.BlockSpec((1, N, D), lambda b, e: (b, e, 0))
    wspec = pl.BlockSpec((D, D), lambda b, e: (0, 0))
    bspec = pl.BlockSpec((D,), lambda b, e: (0,))
    ospec = pl.BlockSpec((1, N, D), lambda b, e: (b, e, 0))
    out_sd = jax.ShapeDtypeStruct((B, T, D), jnp.bfloat16)
    return pl.pallas_call(
        _qkv_body,
        grid=(B, E),
        in_specs=[xspec, wspec, wspec, wspec, bspec, bspec, bspec, bspec, bspec],
        out_specs=[ospec, ospec, ospec],
        out_shape=[out_sd, out_sd, out_sd],
    )(xg_b, q_wb, k_wb, v_wb, q_b, k_b, v_b, ln_w, ln_b)


def _attn_body(q_ref, k_ref, v_ref, o_ref):
    q = q_ref[0]
    k = k_ref[0]
    v = v_ref[0]
    s = jax.lax.dot_general(q, k, (((1,), (1,)), ((), ())),
                            preferred_element_type=jnp.float32)
    p = jnp.exp(s)
    p = p * (1.0 / jnp.sum(p, axis=1, keepdims=True))
    o = jax.lax.dot_general(p.astype(jnp.bfloat16), v, (((1,), (0,)), ((), ())),
                            preferred_element_type=jnp.float32)
    o_ref[0] = o.astype(jnp.bfloat16)


def _attention(q, k, v, bq=2048):
    qspec = pl.BlockSpec((1, bq, DH), lambda b, h, i: (b, i, h))
    kvspec = pl.BlockSpec((1, T, DH), lambda b, h, i: (b, 0, h))
    return pl.pallas_call(
        _attn_body,
        grid=(B, H, T // bq),
        in_specs=[qspec, kvspec, kvspec],
        out_specs=qspec,
        out_shape=jax.ShapeDtypeStruct((B, T, D), jnp.bfloat16),
    )(q, k, v)


def _oproj_body(a_ref, x_ref, ow_ref, ob_ref, o_ref):
    e = pl.program_id(1)
    ab = a_ref[0]
    xb = x_ref[0]
    dn = (((1,), (1,)), ((), ()))
    for i in range(E):
        m = D >> i

        @pl.when(e == i)
        def _(m=m):
            proj = jax.lax.dot_general(ab[:, :m], ow_ref[:m, :m], dn,
                                       preferred_element_type=jnp.float32) + ob_ref[:m]
            if m == D:
                o_ref[0] = xb + proj
            else:
                o_ref[0] = jnp.concatenate([xb[:, :m] + proj, xb[:, m:]], axis=1)


def _oproj_all(attn_g, x_g, o_wb, o_b):
    aspec = pl.BlockSpec((1, N, D), lambda b, e: (b, e, 0))
    wspec = pl.BlockSpec((D, D), lambda b, e: (0, 0))
    bspec = pl.BlockSpec((D,), lambda b, e: (0,))
    return pl.pallas_call(
        _oproj_body,
        grid=(B, E),
        in_specs=[aspec, aspec, wspec, bspec],
        out_specs=aspec,
        out_shape=jax.ShapeDtypeStruct((B, T, D), jnp.float32),
    )(attn_g, x_g, o_wb, o_b)


def kernel(x, router_prob, q_w, q_b, k_w, k_b, v_w, v_b, o_w, o_b, ln_w, ln_b):
    q_wb, k_wb, v_wb, o_wb = _cast_weights(q_w, k_w, v_w, o_w)

    npt = jnp.transpose(-router_prob, (0, 2, 1))
    perm, nnp_t = _greedy_route(npt, negate_out=False)
    xg = jnp.take_along_axis(x, perm[:, :, None], axis=1)

    q, k, v = _qkv_all(xg, q_wb, k_wb, v_wb, q_b, k_b, v_b, ln_w, ln_b)

    attn_out = _attention(q, k, v)

    perm2, fp_t = _greedy_route(nnp_t, negate_out=True)
    attn_g = jnp.take_along_axis(attn_out, perm2[:, :, None], axis=1)
    x_g = jnp.take_along_axis(x, perm2[:, :, None], axis=1)
    final_probs = jnp.transpose(fp_t, (0, 2, 1))

    out = _oproj_all(attn_g, x_g, o_wb, o_b)
    return out, final_probs
